# Initial kernel scaffold; baseline (speedup 1.0000x reference)
import jax
import jax.numpy as jnp
from jax.experimental import pallas as pl


def kernel(node_types, node_labels, edge_labels, edge_index, Wn0, We0, a0, Wm0, Wn1, We1, a1, Wm1, fcW, fcb):
    raise NotImplementedError("write your pallas kernel here")


# speedup vs baseline: 1.2208x; 1.2208x over previous
import jax
import jax.numpy as jnp
from jax.experimental import pallas as pl

N_NODES = 50000


def _fc_kernel(pooled_ref, w_ref, b_ref, out_ref):
    out_ref[...] = pooled_ref[...] @ w_ref[...] + b_ref[...]


def kernel(node_types, node_labels, edge_labels, edge_index, Wn0, We0, a0, Wm0, Wn1, We1, a1, Wm1, fcW, fcb):
    src, dst = edge_index[0], edge_index[1]
    N = N_NODES

    def layer(xn, xe, Wn, We, a, Wm):
        outs_n, ze_cat = [], []
        for h in range(2):
            zn = xn @ Wn[h]
            ze = xe @ We[h]
            ls = zn @ a[h][0:16]
            ld = zn @ a[h][24:40]
            le = ze @ a[h][16:24]
            mn = zn @ Wm[h][0:16]
            me = ze @ Wm[h][16:24]
            l = ls[src] + le + ld[dst]
            l = jnp.maximum(l, 0.2 * l)
            ex = jnp.exp(l)
            denom = jax.ops.segment_sum(ex, dst, num_segments=N)
            m = mn[src] + me
            raw = jax.ops.segment_sum(ex[:, None] * m, dst, num_segments=N)
            out = raw / (denom[:, None] + 1e-9)
            outs_n.append(out)
            ze_cat.append(ze)
        return jnp.concatenate(outs_n, 1), jnp.concatenate(ze_cat, 1)

    xn = jnp.concatenate([node_types, node_labels], 1)
    xe = edge_labels
    xn, xe = layer(xn, xe, Wn0, We0, a0, Wm0)
    xn = jnp.maximum(xn, 0.01 * xn)
    xe = jnp.maximum(xe, 0.01 * xe)
    xn, _ = layer(xn, xe, Wn1, We1, a1, Wm1)
    pooled = jnp.sum(xn, 0, keepdims=True)
    out = pl.pallas_call(
        _fc_kernel,
        out_shape=jax.ShapeDtypeStruct((1, fcW.shape[1]), jnp.float32),
    )(pooled, fcW, fcb[None, :])
    return out


# speedup vs baseline: 5.9694x; 4.8897x over previous
import jax
import jax.numpy as jnp
from jax import lax
from jax.experimental import pallas as pl
from jax.experimental.pallas import tpu as pltpu
from jax.experimental.pallas import tpu_sc as plsc

N = 50000
E = 800000
N_PAD = 50048
NSUB = N_PAD // 16
BN = 3128
BE = 6400
EPB = 64
NS = 16
NC = 2
NBLK_TOTAL = E // EPB


def _node0_body(t_ref, lab_ref, wn_ref, ms_ref, av_ref, ntab_ref, mn_ref):
    xn = jnp.concatenate([t_ref[...], lab_ref[...]], axis=1)
    zn = jnp.dot(xn, wn_ref[...], preferred_element_type=jnp.float32)
    mncat = jnp.dot(zn, ms_ref[...], preferred_element_type=jnp.float32)
    lsld = jnp.dot(zn, av_ref[...], preferred_element_type=jnp.float32)
    z6 = jnp.zeros((lsld.shape[0], 6), jnp.float32)
    ntab_ref[0] = jnp.concatenate([lsld[:, 0:2], z6], axis=1)
    ntab_ref[1] = jnp.concatenate([lsld[:, 2:4], z6], axis=1)
    mn_ref[0] = mncat[:, 0:16]
    mn_ref[1] = mncat[:, 16:32]


def _edge0_body(xe_ref, wec_ref, wme_ref, vle_ref, ze_ref, le_ref, me_ref):
    xe = xe_ref[...]
    ze_ref[...] = jnp.dot(xe, wec_ref[...], preferred_element_type=jnp.float32)
    mecat = jnp.dot(xe, wme_ref[...], preferred_element_type=jnp.float32)
    lecat = jnp.dot(xe, vle_ref[...], preferred_element_type=jnp.float32)
    le_ref[0] = lecat[:, 0]
    le_ref[1] = lecat[:, 1]
    me_ref[0] = mecat[:, 0:16]
    me_ref[1] = mecat[:, 16:32]


def _edge1_body(ze_ref, wme_ref, vle_ref, le_ref, me_ref):
    ze = ze_ref[...]
    xe = jnp.maximum(ze, 0.01 * ze)
    mecat = jnp.dot(xe, wme_ref[...], preferred_element_type=jnp.float32)
    lecat = jnp.dot(xe, vle_ref[...], preferred_element_type=jnp.float32)
    le_ref[0] = lecat[:, 0]
    le_ref[1] = lecat[:, 1]
    me_ref[0] = mecat[:, 0:16]
    me_ref[1] = mecat[:, 16:32]


def _mid_body(acc_ref, den_ref, wn_ref, ms_ref, av_ref, ntab_ref, mn_ref):
    x0 = acc_ref[0] / (den_ref[0][:, 0:1] + 1e-9)
    x1 = acc_ref[1] / (den_ref[1][:, 0:1] + 1e-9)
    x = jnp.concatenate([x0, x1], axis=1)
    x = jnp.maximum(x, 0.01 * x)
    zn = jnp.dot(x, wn_ref[...], preferred_element_type=jnp.float32)
    mncat = jnp.dot(zn, ms_ref[...], preferred_element_type=jnp.float32)
    lsld = jnp.dot(zn, av_ref[...], preferred_element_type=jnp.float32)
    z6 = jnp.zeros((lsld.shape[0], 6), jnp.float32)
    ntab_ref[0] = jnp.concatenate([lsld[:, 0:2], z6], axis=1)
    ntab_ref[1] = jnp.concatenate([lsld[:, 2:4], z6], axis=1)
    mn_ref[0] = mncat[:, 0:16]
    mn_ref[1] = mncat[:, 16:32]


def _final_body(acc_ref, den_ref, fcw_ref, fcb_ref, out_ref, pooled_ref):
    i = pl.program_id(0)

    @pl.when(i == 0)
    def _():
        pooled_ref[...] = jnp.zeros_like(pooled_ref)

    x0 = acc_ref[0] / (den_ref[0][:, 0:1] + 1e-9)
    x1 = acc_ref[1] / (den_ref[1][:, 0:1] + 1e-9)
    x = jnp.concatenate([x0, x1], axis=1)
    pooled_ref[0:1, :] += jnp.sum(x, axis=0, keepdims=True)

    @pl.when(i == pl.num_programs(0) - 1)
    def _():
        out_ref[...] = (
            jnp.dot(pooled_ref[0:1, :], fcw_ref[...],
                    preferred_element_type=jnp.float32) + fcb_ref[...]
        )


def _sc_scalar_body(src_h, dst_h, ntab_h, le_h, ex_out_h, den_out_h,
                    src_v, dst_v, le_v, ex_v, rows_v, lsr_v, ldr_v,
                    den_s, sem, sem2):
    c = lax.axis_index("c")
    s = lax.axis_index("s")

    zv = jnp.zeros((16,), jnp.float32)
    iota = lax.iota(jnp.int32, 16)
    zeros_i = jnp.zeros((16,), jnp.int32)
    ones_i = jnp.ones((16,), jnp.int32)
    for i in range(EPB):
        rows_v[i, 0:16] = zv
    for k in range(48):
        pltpu.sync_copy(rows_v, den_s.at[pl.ds(s * NSUB + k * 64, 64)])
    pltpu.sync_copy(rows_v.at[pl.ds(0, 56)],
                    den_s.at[pl.ds(s * NSUB + 48 * 64, 56)])
    plsc.subcore_barrier()

    nblk = jnp.where(s < NBLK_TOTAL % NS, NBLK_TOTAL // NS + 1,
                     NBLK_TOTAL // NS)

    def blk(b, carry):
        start = (b * NS + s) * EPB
        pltpu.sync_copy(src_h.at[pl.ds(start, EPB)], src_v)
        pltpu.sync_copy(dst_h.at[pl.ds(start, EPB)], dst_v)
        pltpu.sync_copy(le_h.at[c].at[pl.ds(start, EPB)], le_v)
        pltpu.async_copy(ntab_h.at[c].at[src_v], lsr_v, sem).wait()
        pltpu.async_copy(ntab_h.at[c].at[dst_v], ldr_v, sem2).wait()
        for i in range(EPB // 16):
            sl = pl.ds(i * 16, 16)
            ridx = jnp.full((16,), i * 16, jnp.int32) + iota
            ls = plsc.load_gather(lsr_v, [ridx, zeros_i])
            ld = plsc.load_gather(ldr_v, [ridx, ones_i])
            l = ls + ld + le_v[sl]
            l = jnp.maximum(l, 0.2 * l)
            ex = jnp.exp(l)
            ex_v[sl] = ex
        pltpu.sync_copy(ex_v, ex_out_h.at[c].at[pl.ds(start, EPB)])

        def rb(e, cc):
            exb = plsc.load_gather(ex_v, [jnp.full((16,), e, jnp.int32)])
            rows_v[e, 0:16] = jnp.where(iota == 0, exb, 0.0)
            return cc

        lax.fori_loop(0, EPB, rb, 0)
        pltpu.sync_copy(rows_v, den_s.at[dst_v], add=True)
        return carry

    lax.fori_loop(0, nblk, blk, 0)
    plsc.subcore_barrier()
    for k in range(48):
        base = s * NSUB + k * 64
        pltpu.sync_copy(den_s.at[pl.ds(base, 64)], rows_v)
        pltpu.sync_copy(rows_v, den_out_h.at[c].at[pl.ds(base, 64)])
    base = s * NSUB + 48 * 64
    pltpu.sync_copy(den_s.at[pl.ds(base, 56)], rows_v.at[pl.ds(0, 56)])
    pltpu.sync_copy(rows_v.at[pl.ds(0, 56)],
                    den_out_h.at[c].at[pl.ds(base, 56)])


def _sc_row_body(src_h, dst_h, ex_h, me_h, mn_h, acc_out_h,
                 src_v, dst_v, ex_v, me_v, mnrows_v, rows_v, acc_s, sem):
    c = lax.axis_index("c")
    s = lax.axis_index("s")

    zv = jnp.zeros((16,), jnp.float32)
    for i in range(EPB):
        rows_v[i, 0:16] = zv
    for k in range(48):
        pltpu.sync_copy(rows_v, acc_s.at[pl.ds(s * NSUB + k * 64, 64)])
    pltpu.sync_copy(rows_v.at[pl.ds(0, 56)],
                    acc_s.at[pl.ds(s * NSUB + 48 * 64, 56)])
    plsc.subcore_barrier()

    nblk = jnp.where(s < NBLK_TOTAL % NS, NBLK_TOTAL // NS + 1,
                     NBLK_TOTAL // NS)

    def blk(b, carry):
        start = (b * NS + s) * EPB
        pltpu.sync_copy(src_h.at[pl.ds(start, EPB)], src_v)
        pltpu.sync_copy(dst_h.at[pl.ds(start, EPB)], dst_v)
        pltpu.sync_copy(ex_h.at[c].at[pl.ds(start, EPB)], ex_v)
        for q in range(EPB // 32):
            pltpu.sync_copy(me_h.at[c].at[pl.ds(start + q * 32, 32)],
                            me_v.at[pl.ds(q * 32, 32)])
        pltpu.async_copy(mn_h.at[c].at[src_v], mnrows_v, sem).wait()

        def rb(e, cc):
            exb = plsc.load_gather(ex_v, [jnp.full((16,), e, jnp.int32)])
            rows_v[e, 0:16] = exb * (mnrows_v[e, 0:16] + me_v[e, 0:16])
            return cc

        lax.fori_loop(0, EPB, rb, 0)
        pltpu.sync_copy(rows_v, acc_s.at[dst_v], add=True)
        return carry

    lax.fori_loop(0, nblk, blk, 0)
    plsc.subcore_barrier()
    for k in range(48):
        base = s * NSUB + k * 64
        pltpu.sync_copy(acc_s.at[pl.ds(base, 64)], rows_v)
        pltpu.sync_copy(rows_v, acc_out_h.at[c].at[pl.ds(base, 64)])
    base = s * NSUB + 48 * 64
    pltpu.sync_copy(acc_s.at[pl.ds(base, 56)], rows_v.at[pl.ds(0, 56)])
    pltpu.sync_copy(rows_v.at[pl.ds(0, 56)],
                    acc_out_h.at[c].at[pl.ds(base, 56)])


def _sc_edge_pass(src, dst, ntab, mn, le, me):
    mesh = plsc.VectorSubcoreMesh(core_axis_name="c", subcore_axis_name="s",
                                  num_cores=NC, num_subcores=NS)
    ex, den = pl.kernel(
        _sc_scalar_body,
        out_type=(
            jax.ShapeDtypeStruct((NC, E), jnp.float32),
            jax.ShapeDtypeStruct((NC, N_PAD, 16), jnp.float32),
        ),
        mesh=mesh,
        compiler_params=pltpu.CompilerParams(needs_layout_passes=False, use_tc_tiling_on_sc=False),
        scratch_types=[
            pltpu.VMEM((EPB,), jnp.int32),
            pltpu.VMEM((EPB,), jnp.int32),
            pltpu.VMEM((EPB,), jnp.float32),
            pltpu.VMEM((EPB,), jnp.float32),
            pltpu.VMEM((EPB, 16), jnp.float32),
            pltpu.VMEM((EPB, 8), jnp.float32),
            pltpu.VMEM((EPB, 8), jnp.float32),
            pltpu.VMEM_SHARED((N_PAD, 16), jnp.float32),
            pltpu.SemaphoreType.DMA,
            pltpu.SemaphoreType.DMA,
        ],
    )(src, dst, ntab, le)

    acc = pl.kernel(
        _sc_row_body,
        out_type=jax.ShapeDtypeStruct((NC, N_PAD, 16), jnp.float32),
        mesh=mesh,
        compiler_params=pltpu.CompilerParams(needs_layout_passes=False, use_tc_tiling_on_sc=False),
        scratch_types=[
            pltpu.VMEM((EPB,), jnp.int32),
            pltpu.VMEM((EPB,), jnp.int32),
            pltpu.VMEM((EPB,), jnp.float32),
            pltpu.VMEM((EPB, 16), jnp.float32),
            pltpu.VMEM((EPB, 16), jnp.float32),
            pltpu.VMEM((EPB, 16), jnp.float32),
            pltpu.VMEM_SHARED((N_PAD, 16), jnp.float32),
            pltpu.SemaphoreType.DMA,
        ],
    )(src, dst, ex, me, mn)
    return acc, den


def _node_weight_mats(Wn, a, Wm):
    wncat = jnp.concatenate([Wn[0], Wn[1]], axis=1)
    ms = jnp.zeros((32, 32), jnp.float32)
    ms = ms.at[0:16, 0:16].set(Wm[0][0:16])
    ms = ms.at[16:32, 16:32].set(Wm[1][0:16])
    av = jnp.zeros((32, 4), jnp.float32)
    av = av.at[0:16, 0].set(a[0][0:16])
    av = av.at[0:16, 1].set(a[0][24:40])
    av = av.at[16:32, 2].set(a[1][0:16])
    av = av.at[16:32, 3].set(a[1][24:40])
    return wncat, ms, av


def _edge_weight_mats(We, a, Wm):
    wme = jnp.concatenate([We[0] @ Wm[0][16:24], We[1] @ Wm[1][16:24]], axis=1)
    vle = jnp.stack([We[0] @ a[0][16:24], We[1] @ a[1][16:24]], axis=1)
    return wme, vle


def kernel(node_types, node_labels, edge_labels, edge_index,
           Wn0, We0, a0, Wm0, Wn1, We1, a1, Wm1, fcW, fcb):
    src = edge_index[0]
    dst = edge_index[1]

    wncat0, ms0, av0 = _node_weight_mats(Wn0, a0, Wm0)
    wncat1, ms1, av1 = _node_weight_mats(Wn1, a1, Wm1)
    wecat0 = jnp.concatenate([We0[0], We0[1]], axis=1)
    wme0, vle0 = _edge_weight_mats(We0, a0, Wm0)
    wme1, vle1 = _edge_weight_mats(We1, a1, Wm1)

    ngrid = N_PAD // BN
    egrid = E // BE
    full = lambda shape: pl.BlockSpec(shape, lambda i: tuple(0 for _ in shape))

    ntab0, mn0 = pl.pallas_call(
        _node0_body,
        grid=(ngrid,),
        in_specs=[
            pl.BlockSpec((BN, 32), lambda i: (i, 0)),
            pl.BlockSpec((BN, 64), lambda i: (i, 0)),
            full((96, 32)), full((32, 32)), full((32, 4)),
        ],
        out_specs=[
            pl.BlockSpec((2, BN, 8), lambda i: (0, i, 0)),
            pl.BlockSpec((2, BN, 16), lambda i: (0, i, 0)),
        ],
        out_shape=[
            jax.ShapeDtypeStruct((2, N_PAD, 8), jnp.float32),
            jax.ShapeDtypeStruct((2, N_PAD, 16), jnp.float32),
        ],
    )(node_types, node_labels, wncat0, ms0, av0)

    zecat, le0, me0 = pl.pallas_call(
        _edge0_body,
        grid=(egrid,),
        in_specs=[
            pl.BlockSpec((BE, 16), lambda i: (i, 0)),
            full((16, 16)), full((16, 32)), full((16, 2)),
        ],
        out_specs=[
            pl.BlockSpec((BE, 16), lambda i: (i, 0)),
            pl.BlockSpec((2, BE), lambda i: (0, i)),
            pl.BlockSpec((2, BE, 16), lambda i: (0, i, 0)),
        ],
        out_shape=[
            jax.ShapeDtypeStruct((E, 16), jnp.float32),
            jax.ShapeDtypeStruct((2, E), jnp.float32),
            jax.ShapeDtypeStruct((2, E, 16), jnp.float32),
        ],
    )(edge_labels, wecat0, wme0, vle0)

    acc0, den0 = _sc_edge_pass(src, dst, ntab0, mn0, le0, me0)

    ntab1, mn1 = pl.pallas_call(
        _mid_body,
        grid=(ngrid,),
        in_specs=[
            pl.BlockSpec((2, BN, 16), lambda i: (0, i, 0)),
            pl.BlockSpec((2, BN, 16), lambda i: (0, i, 0)),
            full((32, 32)), full((32, 32)), full((32, 4)),
        ],
        out_specs=[
            pl.BlockSpec((2, BN, 8), lambda i: (0, i, 0)),
            pl.BlockSpec((2, BN, 16), lambda i: (0, i, 0)),
        ],
        out_shape=[
            jax.ShapeDtypeStruct((2, N_PAD, 8), jnp.float32),
            jax.ShapeDtypeStruct((2, N_PAD, 16), jnp.float32),
        ],
    )(acc0, den0, wncat1, ms1, av1)

    le1, me1 = pl.pallas_call(
        _edge1_body,
        grid=(egrid,),
        in_specs=[
            pl.BlockSpec((BE, 16), lambda i: (i, 0)),
            full((16, 32)), full((16, 2)),
        ],
        out_specs=[
            pl.BlockSpec((2, BE), lambda i: (0, i)),
            pl.BlockSpec((2, BE, 16), lambda i: (0, i, 0)),
        ],
        out_shape=[
            jax.ShapeDtypeStruct((2, E), jnp.float32),
            jax.ShapeDtypeStruct((2, E, 16), jnp.float32),
        ],
    )(zecat, wme1, vle1)

    acc1, den1 = _sc_edge_pass(src, dst, ntab1, mn1, le1, me1)

    out = pl.pallas_call(
        _final_body,
        grid=(ngrid,),
        in_specs=[
            pl.BlockSpec((2, BN, 16), lambda i: (0, i, 0)),
            pl.BlockSpec((2, BN, 16), lambda i: (0, i, 0)),
            full((32, 10)), full((1, 10)),
        ],
        out_specs=pl.BlockSpec((1, 10), lambda i: (0, 0)),
        out_shape=jax.ShapeDtypeStruct((1, 10), jnp.float32),
        scratch_shapes=[pltpu.VMEM((8, 32), jnp.float32)],
    )(acc1, den1, fcW, fcb[None, :])
    return out


# speedup vs baseline: 18.2264x; 3.0533x over previous
import jax
import jax.numpy as jnp
from jax import lax
from jax.experimental import pallas as pl
from jax.experimental.pallas import tpu as pltpu
from jax.experimental.pallas import tpu_sc as plsc

N = 50000
E = 800000
N_PAD = 50048
NSUB = N_PAD // 16
BN = 3128
BE = 6400
EPB = 128
NS = 16
NC = 2
NBLK_TOTAL = E // EPB


def _node0_body(t_ref, lab_ref, wn_ref, ms_ref, av_ref, ntab_ref, mn_ref):
    xn = jnp.concatenate([t_ref[...], lab_ref[...]], axis=1)
    zn = jnp.dot(xn, wn_ref[...], preferred_element_type=jnp.float32)
    mncat = jnp.dot(zn, ms_ref[...], preferred_element_type=jnp.float32)
    lsld = jnp.dot(zn, av_ref[...], preferred_element_type=jnp.float32)
    z6 = jnp.zeros((lsld.shape[0], 6), jnp.float32)
    ntab_ref[0] = jnp.concatenate([lsld[:, 0:2], z6], axis=1)
    ntab_ref[1] = jnp.concatenate([lsld[:, 2:4], z6], axis=1)
    mn_ref[0] = mncat[:, 0:16]
    mn_ref[1] = mncat[:, 16:32]


def _edge0_body(xe_ref, wec_ref, wme_ref, vle_ref, ze_ref, le_ref, me_ref):
    xe = xe_ref[...]
    ze_ref[...] = jnp.dot(xe, wec_ref[...], preferred_element_type=jnp.float32)
    mecat = jnp.dot(xe, wme_ref[...], preferred_element_type=jnp.float32)
    lecat = jnp.dot(xe, vle_ref[...], preferred_element_type=jnp.float32)
    le_ref[0] = lecat[:, 0]
    le_ref[1] = lecat[:, 1]
    me_ref[0] = mecat[:, 0:16]
    me_ref[1] = mecat[:, 16:32]


def _edge1_body(ze_ref, wme_ref, vle_ref, le_ref, me_ref):
    ze = ze_ref[...]
    xe = jnp.maximum(ze, 0.01 * ze)
    mecat = jnp.dot(xe, wme_ref[...], preferred_element_type=jnp.float32)
    lecat = jnp.dot(xe, vle_ref[...], preferred_element_type=jnp.float32)
    le_ref[0] = lecat[:, 0]
    le_ref[1] = lecat[:, 1]
    me_ref[0] = mecat[:, 0:16]
    me_ref[1] = mecat[:, 16:32]


def _mid_body(acc_ref, wn_ref, ms_ref, av_ref, ntab_ref, mn_ref):
    r0 = acc_ref[0]
    r1 = acc_ref[1]
    x0 = r0[:, 0:16] / (r0[:, 16:17] + 1e-9)
    x1 = r1[:, 0:16] / (r1[:, 16:17] + 1e-9)
    x = jnp.concatenate([x0, x1], axis=1)
    x = jnp.maximum(x, 0.01 * x)
    zn = jnp.dot(x, wn_ref[...], preferred_element_type=jnp.float32)
    mncat = jnp.dot(zn, ms_ref[...], preferred_element_type=jnp.float32)
    lsld = jnp.dot(zn, av_ref[...], preferred_element_type=jnp.float32)
    z6 = jnp.zeros((lsld.shape[0], 6), jnp.float32)
    ntab_ref[0] = jnp.concatenate([lsld[:, 0:2], z6], axis=1)
    ntab_ref[1] = jnp.concatenate([lsld[:, 2:4], z6], axis=1)
    mn_ref[0] = mncat[:, 0:16]
    mn_ref[1] = mncat[:, 16:32]


def _final_body(acc_ref, fcw_ref, fcb_ref, out_ref, pooled_ref):
    i = pl.program_id(0)

    @pl.when(i == 0)
    def _():
        pooled_ref[...] = jnp.zeros_like(pooled_ref)

    r0 = acc_ref[0]
    r1 = acc_ref[1]
    x0 = r0[:, 0:16] / (r0[:, 16:17] + 1e-9)
    x1 = r1[:, 0:16] / (r1[:, 16:17] + 1e-9)
    x = jnp.concatenate([x0, x1], axis=1)
    pooled_ref[0:1, :] += jnp.sum(x, axis=0, keepdims=True)

    @pl.when(i == pl.num_programs(0) - 1)
    def _():
        out_ref[...] = (
            jnp.dot(pooled_ref[0:1, :], fcw_ref[...],
                    preferred_element_type=jnp.float32) + fcb_ref[...]
        )


def _sc_body(src_h, dst_h, ntab_h, mn_h, le_h, me_h, acc_out_h,
             src_v, dst_v, le_v, ex_v, rows_v, lsr_v, ldr_v, mnrows_v, me_v,
             acc_s, sem_a, sem_b, sem_c, sem_d, sem_e, sem_f, sem_g):
    c = lax.axis_index("c")
    s = lax.axis_index("s")

    zv = jnp.zeros((16,), jnp.float32)
    iota = lax.iota(jnp.int32, 16)
    zeros_i = jnp.zeros((16,), jnp.int32)
    ones_i = jnp.ones((16,), jnp.int32)
    col16 = jnp.full((16,), 16, jnp.int32)
    for i in range(EPB):
        rows_v[i, 8:24] = zv
        rows_v[i, 0:16] = zv
    for k in range(24):
        pltpu.sync_copy(rows_v, acc_s.at[pl.ds(s * NSUB + k * 128, 128)])
    pltpu.sync_copy(rows_v.at[pl.ds(0, 56)],
                    acc_s.at[pl.ds(s * NSUB + 24 * 128, 56)])
    plsc.subcore_barrier()

    nblk = jnp.where(s < NBLK_TOTAL % NS, NBLK_TOTAL // NS + 1,
                     NBLK_TOTAL // NS)

    def blk(b, carry):
        start = (b * NS + s) * EPB
        a_src = pltpu.async_copy(src_h.at[pl.ds(start, EPB)], src_v, sem_a)
        a_dst = pltpu.async_copy(dst_h.at[pl.ds(start, EPB)], dst_v, sem_b)
        a_le = pltpu.async_copy(le_h.at[c].at[pl.ds(start, EPB)], le_v, sem_c)
        a_me = pltpu.async_copy(me_h.at[c].at[pl.ds(start, EPB)], me_v, sem_d)
        a_src.wait()
        a_dst.wait()
        g_mn = pltpu.async_copy(mn_h.at[c].at[src_v], mnrows_v, sem_g)
        g_ls = pltpu.async_copy(ntab_h.at[c].at[src_v], lsr_v, sem_e)
        g_ld = pltpu.async_copy(ntab_h.at[c].at[dst_v], ldr_v, sem_f)
        g_ls.wait()
        g_ld.wait()
        a_le.wait()
        for i in range(EPB // 16):
            sl = pl.ds(i * 16, 16)
            ridx = jnp.full((16,), i * 16, jnp.int32) + iota
            ls = plsc.load_gather(lsr_v, [ridx, zeros_i])
            ld = plsc.load_gather(ldr_v, [ridx, ones_i])
            l = ls + ld + le_v[sl]
            l = jnp.maximum(l, 0.2 * l)
            ex = jnp.exp(l)
            ex_v[sl] = ex
            plsc.store_scatter(rows_v, [ridx, col16], ex)
        a_me.wait()
        g_mn.wait()
        for e in range(EPB):
            exb = plsc.load_gather(ex_v, [jnp.full((16,), e, jnp.int32)])
            rows_v[e, 0:16] = exb * (mnrows_v[e, 0:16] + me_v[e, 0:16])
        pltpu.sync_copy(rows_v, acc_s.at[dst_v], add=True)
        return carry

    lax.fori_loop(0, nblk, blk, 0)
    plsc.subcore_barrier()
    for k in range(24):
        base = s * NSUB + k * 128
        pltpu.sync_copy(acc_s.at[pl.ds(base, 128)], rows_v)
        pltpu.sync_copy(rows_v, acc_out_h.at[c].at[pl.ds(base, 128)])
    base = s * NSUB + 24 * 128
    pltpu.sync_copy(acc_s.at[pl.ds(base, 56)], rows_v.at[pl.ds(0, 56)])
    pltpu.sync_copy(rows_v.at[pl.ds(0, 56)],
                    acc_out_h.at[c].at[pl.ds(base, 56)])


def _sc_edge_pass(src, dst, ntab, mn, le, me):
    mesh = plsc.VectorSubcoreMesh(core_axis_name="c", subcore_axis_name="s",
                                  num_cores=NC, num_subcores=NS)
    return pl.kernel(
        _sc_body,
        out_type=jax.ShapeDtypeStruct((NC, N_PAD, 24), jnp.float32),
        mesh=mesh,
        compiler_params=pltpu.CompilerParams(needs_layout_passes=False,
                                             use_tc_tiling_on_sc=False),
        scratch_types=[
            pltpu.VMEM((EPB,), jnp.int32),
            pltpu.VMEM((EPB,), jnp.int32),
            pltpu.VMEM((EPB,), jnp.float32),
            pltpu.VMEM((EPB,), jnp.float32),
            pltpu.VMEM((EPB, 24), jnp.float32),
            pltpu.VMEM((EPB, 8), jnp.float32),
            pltpu.VMEM((EPB, 8), jnp.float32),
            pltpu.VMEM((EPB, 16), jnp.float32),
            pltpu.VMEM((EPB, 16), jnp.float32),
            pltpu.VMEM_SHARED((N_PAD, 24), jnp.float32),
            pltpu.SemaphoreType.DMA,
            pltpu.SemaphoreType.DMA,
            pltpu.SemaphoreType.DMA,
            pltpu.SemaphoreType.DMA,
            pltpu.SemaphoreType.DMA,
            pltpu.SemaphoreType.DMA,
            pltpu.SemaphoreType.DMA,
        ],
    )(src, dst, ntab, mn, le, me)


def _node_weight_mats(Wn, a, Wm):
    wncat = jnp.concatenate([Wn[0], Wn[1]], axis=1)
    ms = jnp.zeros((32, 32), jnp.float32)
    ms = ms.at[0:16, 0:16].set(Wm[0][0:16])
    ms = ms.at[16:32, 16:32].set(Wm[1][0:16])
    av = jnp.zeros((32, 4), jnp.float32)
    av = av.at[0:16, 0].set(a[0][0:16])
    av = av.at[0:16, 1].set(a[0][24:40])
    av = av.at[16:32, 2].set(a[1][0:16])
    av = av.at[16:32, 3].set(a[1][24:40])
    return wncat, ms, av


def _edge_weight_mats(We, a, Wm):
    wme = jnp.concatenate([We[0] @ Wm[0][16:24], We[1] @ Wm[1][16:24]], axis=1)
    vle = jnp.stack([We[0] @ a[0][16:24], We[1] @ a[1][16:24]], axis=1)
    return wme, vle


def kernel(node_types, node_labels, edge_labels, edge_index,
           Wn0, We0, a0, Wm0, Wn1, We1, a1, Wm1, fcW, fcb):
    src = edge_index[0]
    dst = edge_index[1]

    wncat0, ms0, av0 = _node_weight_mats(Wn0, a0, Wm0)
    wncat1, ms1, av1 = _node_weight_mats(Wn1, a1, Wm1)
    wecat0 = jnp.concatenate([We0[0], We0[1]], axis=1)
    wme0, vle0 = _edge_weight_mats(We0, a0, Wm0)
    wme1, vle1 = _edge_weight_mats(We1, a1, Wm1)

    ngrid = N_PAD // BN
    egrid = E // BE
    full = lambda shape: pl.BlockSpec(shape, lambda i: tuple(0 for _ in shape))

    ntab0, mn0 = pl.pallas_call(
        _node0_body,
        grid=(ngrid,),
        in_specs=[
            pl.BlockSpec((BN, 32), lambda i: (i, 0)),
            pl.BlockSpec((BN, 64), lambda i: (i, 0)),
            full((96, 32)), full((32, 32)), full((32, 4)),
        ],
        out_specs=[
            pl.BlockSpec((2, BN, 8), lambda i: (0, i, 0)),
            pl.BlockSpec((2, BN, 16), lambda i: (0, i, 0)),
        ],
        out_shape=[
            jax.ShapeDtypeStruct((2, N_PAD, 8), jnp.float32),
            jax.ShapeDtypeStruct((2, N_PAD, 16), jnp.float32),
        ],
    )(node_types, node_labels, wncat0, ms0, av0)

    zecat, le0, me0 = pl.pallas_call(
        _edge0_body,
        grid=(egrid,),
        in_specs=[
            pl.BlockSpec((BE, 16), lambda i: (i, 0)),
            full((16, 16)), full((16, 32)), full((16, 2)),
        ],
        out_specs=[
            pl.BlockSpec((BE, 16), lambda i: (i, 0)),
            pl.BlockSpec((2, BE), lambda i: (0, i)),
            pl.BlockSpec((2, BE, 16), lambda i: (0, i, 0)),
        ],
        out_shape=[
            jax.ShapeDtypeStruct((E, 16), jnp.float32),
            jax.ShapeDtypeStruct((2, E), jnp.float32),
            jax.ShapeDtypeStruct((2, E, 16), jnp.float32),
        ],
    )(edge_labels, wecat0, wme0, vle0)

    acc0 = _sc_edge_pass(src, dst, ntab0, mn0, le0, me0)

    ntab1, mn1 = pl.pallas_call(
        _mid_body,
        grid=(ngrid,),
        in_specs=[
            pl.BlockSpec((2, BN, 24), lambda i: (0, i, 0)),
            full((32, 32)), full((32, 32)), full((32, 4)),
        ],
        out_specs=[
            pl.BlockSpec((2, BN, 8), lambda i: (0, i, 0)),
            pl.BlockSpec((2, BN, 16), lambda i: (0, i, 0)),
        ],
        out_shape=[
            jax.ShapeDtypeStruct((2, N_PAD, 8), jnp.float32),
            jax.ShapeDtypeStruct((2, N_PAD, 16), jnp.float32),
        ],
    )(acc0, wncat1, ms1, av1)

    le1, me1 = pl.pallas_call(
        _edge1_body,
        grid=(egrid,),
        in_specs=[
            pl.BlockSpec((BE, 16), lambda i: (i, 0)),
            full((16, 32)), full((16, 2)),
        ],
        out_specs=[
            pl.BlockSpec((2, BE), lambda i: (0, i)),
            pl.BlockSpec((2, BE, 16), lambda i: (0, i, 0)),
        ],
        out_shape=[
            jax.ShapeDtypeStruct((2, E), jnp.float32),
            jax.ShapeDtypeStruct((2, E, 16), jnp.float32),
        ],
    )(zecat, wme1, vle1)

    acc1 = _sc_edge_pass(src, dst, ntab1, mn1, le1, me1)

    out = pl.pallas_call(
        _final_body,
        grid=(ngrid,),
        in_specs=[
            pl.BlockSpec((2, BN, 24), lambda i: (0, i, 0)),
            full((32, 10)), full((1, 10)),
        ],
        out_specs=pl.BlockSpec((1, 10), lambda i: (0, 0)),
        out_shape=jax.ShapeDtypeStruct((1, 10), jnp.float32),
        scratch_shapes=[pltpu.VMEM((8, 32), jnp.float32)],
    )(acc1, fcW, fcb[None, :])
    return out


# speedup vs baseline: 18.3374x; 1.0061x over previous
import jax
import jax.numpy as jnp
from jax import lax
from jax.experimental import pallas as pl
from jax.experimental.pallas import tpu as pltpu
from jax.experimental.pallas import tpu_sc as plsc

N = 50000
E = 800000
N_PAD = 50048
NSUB = N_PAD // 16
BN = 3128
BE = 6400
EPB = 128
NS = 16
NC = 2
NBLK_TOTAL = E // EPB


def _node0_body(t_ref, lab_ref, wn_ref, ms_ref, av_ref, ntab_ref, mn_ref):
    xn = jnp.concatenate([t_ref[...], lab_ref[...]], axis=1)
    zn = jnp.dot(xn, wn_ref[...], preferred_element_type=jnp.float32)
    mncat = jnp.dot(zn, ms_ref[...], preferred_element_type=jnp.float32)
    lsld = jnp.dot(zn, av_ref[...], preferred_element_type=jnp.float32)
    z6 = jnp.zeros((lsld.shape[0], 6), jnp.float32)
    z7 = jnp.zeros((lsld.shape[0], 7), jnp.float32)
    ntab_ref[0] = jnp.concatenate([lsld[:, 0:2], z6], axis=1)
    ntab_ref[1] = jnp.concatenate([lsld[:, 2:4], z6], axis=1)
    mn_ref[0] = jnp.concatenate([mncat[:, 0:16], lsld[:, 0:1], z7], axis=1)
    mn_ref[1] = jnp.concatenate([mncat[:, 16:32], lsld[:, 2:3], z7], axis=1)


def _edge0_body(xe_ref, wec_ref, wme_ref, vle_ref, ze_ref, me_ref):
    xe = xe_ref[...]
    ze_ref[...] = jnp.dot(xe, wec_ref[...], preferred_element_type=jnp.float32)
    mecat = jnp.dot(xe, wme_ref[...], preferred_element_type=jnp.float32)
    lecat = jnp.dot(xe, vle_ref[...], preferred_element_type=jnp.float32)
    z7 = jnp.zeros((lecat.shape[0], 7), jnp.float32)
    me_ref[0] = jnp.concatenate([mecat[:, 0:16], lecat[:, 0:1], z7], axis=1)
    me_ref[1] = jnp.concatenate([mecat[:, 16:32], lecat[:, 1:2], z7], axis=1)


def _edge1_body(ze_ref, wme_ref, vle_ref, me_ref):
    ze = ze_ref[...]
    xe = jnp.maximum(ze, 0.01 * ze)
    mecat = jnp.dot(xe, wme_ref[...], preferred_element_type=jnp.float32)
    lecat = jnp.dot(xe, vle_ref[...], preferred_element_type=jnp.float32)
    z7 = jnp.zeros((lecat.shape[0], 7), jnp.float32)
    me_ref[0] = jnp.concatenate([mecat[:, 0:16], lecat[:, 0:1], z7], axis=1)
    me_ref[1] = jnp.concatenate([mecat[:, 16:32], lecat[:, 1:2], z7], axis=1)


def _mid_body(acc_ref, wn_ref, ms_ref, av_ref, ntab_ref, mn_ref):
    r0 = acc_ref[0]
    r1 = acc_ref[1]
    x0 = r0[:, 0:16] / (r0[:, 16:17] + 1e-9)
    x1 = r1[:, 0:16] / (r1[:, 16:17] + 1e-9)
    x = jnp.concatenate([x0, x1], axis=1)
    x = jnp.maximum(x, 0.01 * x)
    zn = jnp.dot(x, wn_ref[...], preferred_element_type=jnp.float32)
    mncat = jnp.dot(zn, ms_ref[...], preferred_element_type=jnp.float32)
    lsld = jnp.dot(zn, av_ref[...], preferred_element_type=jnp.float32)
    z6 = jnp.zeros((lsld.shape[0], 6), jnp.float32)
    z7 = jnp.zeros((lsld.shape[0], 7), jnp.float32)
    ntab_ref[0] = jnp.concatenate([lsld[:, 0:2], z6], axis=1)
    ntab_ref[1] = jnp.concatenate([lsld[:, 2:4], z6], axis=1)
    mn_ref[0] = jnp.concatenate([mncat[:, 0:16], lsld[:, 0:1], z7], axis=1)
    mn_ref[1] = jnp.concatenate([mncat[:, 16:32], lsld[:, 2:3], z7], axis=1)


def _final_body(acc_ref, fcw_ref, fcb_ref, out_ref, pooled_ref):
    i = pl.program_id(0)

    @pl.when(i == 0)
    def _():
        pooled_ref[...] = jnp.zeros_like(pooled_ref)

    r0 = acc_ref[0]
    r1 = acc_ref[1]
    x0 = r0[:, 0:16] / (r0[:, 16:17] + 1e-9)
    x1 = r1[:, 0:16] / (r1[:, 16:17] + 1e-9)
    x = jnp.concatenate([x0, x1], axis=1)
    pooled_ref[0:1, :] += jnp.sum(x, axis=0, keepdims=True)

    @pl.when(i == pl.num_programs(0) - 1)
    def _():
        out_ref[...] = (
            jnp.dot(pooled_ref[0:1, :], fcw_ref[...],
                    preferred_element_type=jnp.float32) + fcb_ref[...]
        )


def _sc_body(src_h, dst_h, ntab_h, mn_h, me_h, acc_out_h,
             src_v, dst_v, ex_v, rows_v, ldr_v, mnls_v, me_v,
             acc_s, sem_a, sem_b, sem_d, sem_f, sem_g, sem_s):
    c = lax.axis_index("c")
    s = lax.axis_index("s")

    zv = jnp.zeros((16,), jnp.float32)
    iota = lax.iota(jnp.int32, 16)
    zeros_i = jnp.zeros((16,), jnp.int32)
    ones_i = jnp.ones((16,), jnp.int32)
    col16 = jnp.full((16,), 16, jnp.int32)
    for i in range(EPB):
        rows_v[i, 8:24] = zv
        rows_v[i, 0:16] = zv
    for i in range(EPB // 16):
        dst_v[pl.ds(i * 16, 16)] = zeros_i
    for k in range(24):
        pltpu.sync_copy(rows_v, acc_s.at[pl.ds(s * NSUB + k * 128, 128)])
    pltpu.sync_copy(rows_v.at[pl.ds(0, 56)],
                    acc_s.at[pl.ds(s * NSUB + 24 * 128, 56)])
    plsc.subcore_barrier()

    nblk = jnp.where(s < NBLK_TOTAL % NS, NBLK_TOTAL // NS + 1,
                     NBLK_TOTAL // NS)

    pltpu.async_copy(rows_v, acc_s.at[dst_v], sem_s, add=True)

    def blk(b, carry):
        start = (b * NS + s) * EPB
        a_src = pltpu.async_copy(src_h.at[pl.ds(start, EPB)], src_v, sem_a)
        a_me = pltpu.async_copy(me_h.at[c].at[pl.ds(start, EPB)], me_v, sem_d)
        pltpu.make_async_copy(rows_v, acc_s.at[dst_v], sem_s).wait()
        a_dst = pltpu.async_copy(dst_h.at[pl.ds(start, EPB)], dst_v, sem_b)
        a_src.wait()
        g_mn = pltpu.async_copy(mn_h.at[c].at[src_v], mnls_v, sem_g)
        a_dst.wait()
        g_ld = pltpu.async_copy(ntab_h.at[c].at[dst_v], ldr_v, sem_f)
        a_me.wait()
        g_mn.wait()
        g_ld.wait()
        for i in range(EPB // 16):
            sl = pl.ds(i * 16, 16)
            ridx = jnp.full((16,), i * 16, jnp.int32) + iota
            ls = plsc.load_gather(mnls_v, [ridx, col16])
            ld = plsc.load_gather(ldr_v, [ridx, ones_i])
            le = plsc.load_gather(me_v, [ridx, col16])
            l = ls + ld + le
            l = jnp.maximum(l, 0.2 * l)
            ex = jnp.exp(l)
            ex_v[sl] = ex
            plsc.store_scatter(rows_v, [ridx, col16], ex)
        for e in range(EPB):
            exb = plsc.load_gather(ex_v, [jnp.full((16,), e, jnp.int32)])
            rows_v[e, 0:16] = exb * (mnls_v[e, 0:16] + me_v[e, 0:16])
        pltpu.async_copy(rows_v, acc_s.at[dst_v], sem_s, add=True)
        return carry

    lax.fori_loop(0, nblk, blk, 0)
    pltpu.make_async_copy(rows_v, acc_s.at[dst_v], sem_s).wait()
    plsc.subcore_barrier()
    for k in range(24):
        base = s * NSUB + k * 128
        pltpu.sync_copy(acc_s.at[pl.ds(base, 128)], rows_v)
        pltpu.sync_copy(rows_v, acc_out_h.at[c].at[pl.ds(base, 128)])
    base = s * NSUB + 24 * 128
    pltpu.sync_copy(acc_s.at[pl.ds(base, 56)], rows_v.at[pl.ds(0, 56)])
    pltpu.sync_copy(rows_v.at[pl.ds(0, 56)],
                    acc_out_h.at[c].at[pl.ds(base, 56)])


def _sc_edge_pass(src, dst, ntab, mn, me):
    mesh = plsc.VectorSubcoreMesh(core_axis_name="c", subcore_axis_name="s",
                                  num_cores=NC, num_subcores=NS)
    return pl.kernel(
        _sc_body,
        out_type=jax.ShapeDtypeStruct((NC, N_PAD, 24), jnp.float32),
        mesh=mesh,
        compiler_params=pltpu.CompilerParams(needs_layout_passes=False,
                                             use_tc_tiling_on_sc=False),
        scratch_types=[
            pltpu.VMEM((EPB,), jnp.int32),
            pltpu.VMEM((EPB,), jnp.int32),
            pltpu.VMEM((EPB,), jnp.float32),
            pltpu.VMEM((EPB, 24), jnp.float32),
            pltpu.VMEM((EPB, 8), jnp.float32),
            pltpu.VMEM((EPB, 24), jnp.float32),
            pltpu.VMEM((EPB, 24), jnp.float32),
            pltpu.VMEM_SHARED((N_PAD, 24), jnp.float32),
            pltpu.SemaphoreType.DMA,
            pltpu.SemaphoreType.DMA,
            pltpu.SemaphoreType.DMA,
            pltpu.SemaphoreType.DMA,
            pltpu.SemaphoreType.DMA,
            pltpu.SemaphoreType.DMA,
        ],
    )(src, dst, ntab, mn, me)


def _node_weight_mats(Wn, a, Wm):
    wncat = jnp.concatenate([Wn[0], Wn[1]], axis=1)
    ms = jnp.zeros((32, 32), jnp.float32)
    ms = ms.at[0:16, 0:16].set(Wm[0][0:16])
    ms = ms.at[16:32, 16:32].set(Wm[1][0:16])
    av = jnp.zeros((32, 4), jnp.float32)
    av = av.at[0:16, 0].set(a[0][0:16])
    av = av.at[0:16, 1].set(a[0][24:40])
    av = av.at[16:32, 2].set(a[1][0:16])
    av = av.at[16:32, 3].set(a[1][24:40])
    return wncat, ms, av


def _edge_weight_mats(We, a, Wm):
    wme = jnp.concatenate([We[0] @ Wm[0][16:24], We[1] @ Wm[1][16:24]], axis=1)
    vle = jnp.stack([We[0] @ a[0][16:24], We[1] @ a[1][16:24]], axis=1)
    return wme, vle


def kernel(node_types, node_labels, edge_labels, edge_index,
           Wn0, We0, a0, Wm0, Wn1, We1, a1, Wm1, fcW, fcb):
    src = edge_index[0]
    dst = edge_index[1]

    wncat0, ms0, av0 = _node_weight_mats(Wn0, a0, Wm0)
    wncat1, ms1, av1 = _node_weight_mats(Wn1, a1, Wm1)
    wecat0 = jnp.concatenate([We0[0], We0[1]], axis=1)
    wme0, vle0 = _edge_weight_mats(We0, a0, Wm0)
    wme1, vle1 = _edge_weight_mats(We1, a1, Wm1)

    ngrid = N_PAD // BN
    egrid = E // BE
    full = lambda shape: pl.BlockSpec(shape, lambda i: tuple(0 for _ in shape))

    ntab0, mn0 = pl.pallas_call(
        _node0_body,
        grid=(ngrid,),
        in_specs=[
            pl.BlockSpec((BN, 32), lambda i: (i, 0)),
            pl.BlockSpec((BN, 64), lambda i: (i, 0)),
            full((96, 32)), full((32, 32)), full((32, 4)),
        ],
        out_specs=[
            pl.BlockSpec((2, BN, 8), lambda i: (0, i, 0)),
            pl.BlockSpec((2, BN, 24), lambda i: (0, i, 0)),
        ],
        out_shape=[
            jax.ShapeDtypeStruct((2, N_PAD, 8), jnp.float32),
            jax.ShapeDtypeStruct((2, N_PAD, 24), jnp.float32),
        ],
    )(node_types, node_labels, wncat0, ms0, av0)

    zecat, me0 = pl.pallas_call(
        _edge0_body,
        grid=(egrid,),
        in_specs=[
            pl.BlockSpec((BE, 16), lambda i: (i, 0)),
            full((16, 16)), full((16, 32)), full((16, 2)),
        ],
        out_specs=[
            pl.BlockSpec((BE, 16), lambda i: (i, 0)),
            pl.BlockSpec((2, BE, 24), lambda i: (0, i, 0)),
        ],
        out_shape=[
            jax.ShapeDtypeStruct((E, 16), jnp.float32),
            jax.ShapeDtypeStruct((2, E, 24), jnp.float32),
        ],
    )(edge_labels, wecat0, wme0, vle0)

    acc0 = _sc_edge_pass(src, dst, ntab0, mn0, me0)

    ntab1, mn1 = pl.pallas_call(
        _mid_body,
        grid=(ngrid,),
        in_specs=[
            pl.BlockSpec((2, BN, 24), lambda i: (0, i, 0)),
            full((32, 32)), full((32, 32)), full((32, 4)),
        ],
        out_specs=[
            pl.BlockSpec((2, BN, 8), lambda i: (0, i, 0)),
            pl.BlockSpec((2, BN, 24), lambda i: (0, i, 0)),
        ],
        out_shape=[
            jax.ShapeDtypeStruct((2, N_PAD, 8), jnp.float32),
            jax.ShapeDtypeStruct((2, N_PAD, 24), jnp.float32),
        ],
    )(acc0, wncat1, ms1, av1)

    (me1,) = pl.pallas_call(
        _edge1_body,
        grid=(egrid,),
        in_specs=[
            pl.BlockSpec((BE, 16), lambda i: (i, 0)),
            full((16, 32)), full((16, 2)),
        ],
        out_specs=[
            pl.BlockSpec((2, BE, 24), lambda i: (0, i, 0)),
        ],
        out_shape=[
            jax.ShapeDtypeStruct((2, E, 24), jnp.float32),
        ],
    )(zecat, wme1, vle1)

    acc1 = _sc_edge_pass(src, dst, ntab1, mn1, me1)

    out = pl.pallas_call(
        _final_body,
        grid=(ngrid,),
        in_specs=[
            pl.BlockSpec((2, BN, 24), lambda i: (0, i, 0)),
            full((32, 10)), full((1, 10)),
        ],
        out_specs=pl.BlockSpec((1, 10), lambda i: (0, 0)),
        out_shape=jax.ShapeDtypeStruct((1, 10), jnp.float32),
        scratch_shapes=[pltpu.VMEM((8, 32), jnp.float32)],
    )(acc1, fcW, fcb[None, :])
    return out
